# Initial kernel scaffold; baseline (speedup 1.0000x reference)
import functools

import jax
import jax.numpy as jnp
from jax import lax
from jax.experimental import pallas as pl
from jax.experimental.pallas import tpu as pltpu
from jax.experimental.pallas import tpu_sc as plsc

NUM_CORES = 2
NUM_SUBCORES = 16
NUM_WORKERS = NUM_CORES * NUM_SUBCORES
CHUNK = 512


@functools.partial(jax.jit, static_argnames=("total", "dim"))
def _gather_flat(flat_idx, table, *, total, dim):
    rows_per_worker = total // NUM_WORKERS
    n_chunks = rows_per_worker // CHUNK
    mesh = plsc.VectorSubcoreMesh(core_axis_name="c", subcore_axis_name="s")

    @functools.partial(
        pl.kernel,
        out_type=jax.ShapeDtypeStruct((total, dim), jnp.float32),
        mesh=mesh,
        scratch_types=[
            pltpu.VMEM((CHUNK,), jnp.int32),
            pltpu.VMEM((CHUNK, dim), jnp.float32),
            pltpu.SemaphoreType.DMA,
        ],
    )
    def k(idx_hbm, table_hbm, out_hbm, idx_v, rows_v, sem):
        wid = lax.axis_index("s") * NUM_CORES + lax.axis_index("c")
        base = wid * rows_per_worker

        def body(g, carry):
            off = base + g * CHUNK
            pltpu.sync_copy(idx_hbm.at[pl.ds(off, CHUNK)], idx_v)
            pltpu.async_copy(table_hbm.at[idx_v], rows_v, sem).wait()
            pltpu.sync_copy(rows_v, out_hbm.at[pl.ds(off, CHUNK)])
            return carry

        lax.fori_loop(0, n_chunks, body, 0)

    return k(flat_idx, table)


def kernel(token_ids, embedding_matrix):
    batch, hist = token_ids.shape
    dim = embedding_matrix.shape[1]
    total = batch * hist
    flat_idx = token_ids.reshape(total).astype(jnp.int32)
    out = _gather_flat(flat_idx, embedding_matrix, total=total, dim=dim)
    return out.reshape(batch, hist, dim)


# speedup vs baseline: 1.7977x; 1.7977x over previous
import functools

import jax
import jax.numpy as jnp
from jax import lax
from jax.experimental import pallas as pl
from jax.experimental.pallas import tpu as pltpu
from jax.experimental.pallas import tpu_sc as plsc

NUM_CORES = 2
NUM_SUBCORES = 16
NUM_WORKERS = NUM_CORES * NUM_SUBCORES
CHUNK = 512


@functools.partial(jax.jit, static_argnames=("total", "dim"))
def _gather_flat(flat_idx, table, *, total, dim):
    rows_per_worker = total // NUM_WORKERS
    n_chunks = rows_per_worker // CHUNK
    mesh = plsc.VectorSubcoreMesh(core_axis_name="c", subcore_axis_name="s")

    @functools.partial(
        pl.kernel,
        out_type=jax.ShapeDtypeStruct((total, dim), jnp.float32),
        mesh=mesh,
        scratch_types=[
            pltpu.VMEM((CHUNK,), jnp.int32),
            pltpu.VMEM((CHUNK, dim), jnp.float32),
            pltpu.SemaphoreType.DMA,
        ],
        compiler_params=pltpu.CompilerParams(use_tc_tiling_on_sc=False),
    )
    def k(idx_hbm, table_hbm, out_hbm, idx_v, rows_v, sem):
        wid = lax.axis_index("s") * NUM_CORES + lax.axis_index("c")
        base = wid * rows_per_worker

        def body(g, carry):
            off = base + g * CHUNK
            pltpu.sync_copy(idx_hbm.at[pl.ds(off, CHUNK)], idx_v)
            pltpu.async_copy(table_hbm.at[idx_v], rows_v, sem).wait()
            pltpu.sync_copy(rows_v, out_hbm.at[pl.ds(off, CHUNK)])
            return carry

        lax.fori_loop(0, n_chunks, body, 0)

    return k(flat_idx, table)


def kernel(token_ids, embedding_matrix):
    batch, hist = token_ids.shape
    dim = embedding_matrix.shape[1]
    total = batch * hist
    flat_idx = token_ids.reshape(total).astype(jnp.int32)
    out = _gather_flat(flat_idx, embedding_matrix, total=total, dim=dim)
    return out.reshape(batch, hist, dim)


# speedup vs baseline: 1.8725x; 1.0416x over previous
import functools

import jax
import jax.numpy as jnp
from jax import lax
from jax.experimental import pallas as pl
from jax.experimental.pallas import tpu as pltpu
from jax.experimental.pallas import tpu_sc as plsc

NUM_CORES = 2
NUM_SUBCORES = 16
NUM_WORKERS = NUM_CORES * NUM_SUBCORES
CHUNK = 800


@functools.partial(jax.jit, static_argnames=("total", "dim"))
def _gather_flat(flat_idx, table, *, total, dim):
    rows_per_worker = total // NUM_WORKERS
    n_chunks = rows_per_worker // CHUNK
    assert n_chunks * CHUNK == rows_per_worker and n_chunks % 2 == 0
    mesh = plsc.VectorSubcoreMesh(core_axis_name="c", subcore_axis_name="s")

    @functools.partial(
        pl.kernel,
        out_type=jax.ShapeDtypeStruct((total, dim), jnp.float32),
        mesh=mesh,
        scratch_types=[
            pltpu.VMEM((2, CHUNK), jnp.int32),
            pltpu.VMEM((2, CHUNK, dim), jnp.float32),
            pltpu.SemaphoreType.DMA,
            pltpu.SemaphoreType.DMA,
            pltpu.SemaphoreType.DMA,
            pltpu.SemaphoreType.DMA,
            pltpu.SemaphoreType.DMA,
            pltpu.SemaphoreType.DMA,
        ],
        compiler_params=pltpu.CompilerParams(use_tc_tiling_on_sc=False),
    )
    def k(idx_hbm, table_hbm, out_hbm, idx_v, rows_v, si0, si1, sg0, sg1,
          so0, so1):
        wid = lax.axis_index("s") * NUM_CORES + lax.axis_index("c")
        base = wid * rows_per_worker
        si = (si0, si1)
        sg = (sg0, sg1)
        so = (so0, so1)

        def i_copy(g, s):
            return pltpu.make_async_copy(
                idx_hbm.at[pl.ds(base + g * CHUNK, CHUNK)], idx_v.at[s],
                si[s])

        def g_copy(s):
            return pltpu.make_async_copy(
                table_hbm.at[idx_v.at[s]], rows_v.at[s], sg[s])

        def s_copy(g, s):
            return pltpu.make_async_copy(
                rows_v.at[s], out_hbm.at[pl.ds(base + g * CHUNK, CHUNK)],
                so[s])

        def step(g, s, prefetch=True):
            o = 1 - s
            g_copy(s).wait()
            s_copy(g, s).start()
            s_copy(g - 1, o).wait()
            i_copy(g + 1, o).wait()
            g_copy(o).start()
            if prefetch:
                i_copy(g + 2, s).start()

        i_copy(0, 0).start()
        i_copy(1, 1).start()
        i_copy(0, 0).wait()
        g_copy(0).start()
        g_copy(0).wait()
        s_copy(0, 0).start()
        i_copy(1, 1).wait()
        g_copy(1).start()
        i_copy(2, 0).start()

        def body(t, carry):
            g = 1 + 2 * t
            step(g, 1)
            step(g + 1, 0)
            return carry

        lax.fori_loop(0, (n_chunks - 4) // 2, body, 0)

        step(n_chunks - 3, 1)
        step(n_chunks - 2, 0, prefetch=False)
        gl = n_chunks - 1
        g_copy(1).wait()
        s_copy(gl, 1).start()
        s_copy(gl - 1, 0).wait()
        s_copy(gl, 1).wait()

    return k(flat_idx, table)


def kernel(token_ids, embedding_matrix):
    batch, hist = token_ids.shape
    dim = embedding_matrix.shape[1]
    total = batch * hist
    flat_idx = token_ids.reshape(total).astype(jnp.int32)
    out = _gather_flat(flat_idx, embedding_matrix, total=total, dim=dim)
    return out.reshape(batch, hist, dim)
